# Initial kernel scaffold; baseline (speedup 1.0000x reference)
import jax
import jax.numpy as jnp
from jax.experimental import pallas as pl


def kernel(hypo_scores, joint_logits, beam_width):
    raise NotImplementedError("write your pallas kernel here")


# speedup vs baseline: 68.3759x; 68.3759x over previous
import jax
import jax.numpy as jnp
from jax.experimental import pallas as pl
from jax.experimental.pallas import tpu as pltpu

_B = 128
_V = 100000
_VN = _V - 1
_C = 1024
_NC = (_V + _C - 1) // _C
_BIG = 2**30


def _pass1_kernel(x_ref, bm_ref, ba_ref, m_ref, s_ref, xp_ref):
    i = pl.program_id(0)
    xp_ref[...] = x_ref[...]
    xt = x_ref[...].T
    col = i * _C + jax.lax.broadcasted_iota(jnp.int32, (_C, _B), 0)
    xn = jnp.where(col < _V, xt, -jnp.inf)
    xc = jnp.where(col < _VN, xt, -jnp.inf)
    bm = jnp.max(xc, axis=0, keepdims=True)
    ba = jnp.min(jnp.where(xc == bm, col, _BIG), axis=0, keepdims=True)
    bm_ref[pl.ds(i, 1), :] = bm
    ba_ref[pl.ds(i, 1), :] = ba

    @pl.when(i == 0)
    def _():
        m_ref[...] = jnp.full((1, _B), -jnp.inf, jnp.float32)
        s_ref[...] = jnp.zeros((1, _B), jnp.float32)

    m_old = m_ref[...]
    s_old = s_ref[...]
    bmax = jnp.max(xn, axis=0, keepdims=True)
    m_new = jnp.maximum(m_old, bmax)
    s_ref[...] = s_old * jnp.exp(m_old - m_new) + jnp.sum(
        jnp.exp(xn - m_new), axis=0, keepdims=True)
    m_ref[...] = m_new


def _topk_kernel(bm_ref, ba_ref, m_ref, s_ref, hypo_ref, joint_ref,
                 os_ref, oh_ref, ot_ref, chunk_ref, sem):
    c = hypo_ref[...] - (m_ref[...] + jnp.log(s_ref[...]))
    cand0 = bm_ref[...] + c
    ba0 = ba_ref[...]
    rank2d = (jax.lax.broadcasted_iota(jnp.int32, (_NC, _B), 1) * _NC
              + jax.lax.broadcasted_iota(jnp.int32, (_NC, _B), 0))
    lane = jax.lax.broadcasted_iota(jnp.int32, (1, _B), 1)
    ccols = jax.lax.broadcasted_iota(jnp.int32, (1, _C), 1)

    def body(k, carry):
        cand, ba, os_, oh, ot = carry
        vmax = jnp.max(cand)
        r = jnp.min(jnp.where(cand == vmax, rank2d, _BIG))
        b = r // _NC
        ch = r % _NC
        colsel = jnp.sum(jnp.where(rank2d == r, ba, 0))
        os_ = jnp.where(lane == k, vmax, os_)
        oh = jnp.where(lane == k, b, oh)
        ot = jnp.where(lane == k, colsel, ot)

        b8 = (b // 8) * 8
        copy = pltpu.make_async_copy(
            joint_ref.at[pl.ds(b8, 8), pl.ds(ch * _C, _C)], chunk_ref, sem)
        copy.start()
        copy.wait()
        cb = jnp.sum(jnp.where(lane == b, c, 0.0))
        cols = ch * _C + ccols
        row = jax.lax.broadcasted_iota(jnp.int32, (8, _C), 0)
        vals = jnp.sum(jnp.where(row == b - b8, chunk_ref[...], 0.0),
                       axis=0, keepdims=True)
        sc = vals + cb
        valid = cols < jnp.minimum((ch + 1) * _C, _VN)
        remaining = valid & ((sc < vmax) | ((sc == vmax) & (cols > colsel)))
        new_bm = jnp.max(jnp.where(remaining, sc, -jnp.inf))
        new_col = jnp.min(jnp.where(remaining & (sc == new_bm), cols, _BIG))
        cand = jnp.where(rank2d == r, new_bm, cand)
        ba = jnp.where(rank2d == r, new_col, ba)
        return cand, ba, os_, oh, ot

    init = (cand0, ba0,
            jnp.zeros((1, _B), jnp.float32),
            jnp.zeros((1, _B), jnp.int32),
            jnp.zeros((1, _B), jnp.int32))
    _, _, os_, oh, ot = jax.lax.fori_loop(0, _B, body, init)
    os_ref[...] = os_
    oh_ref[...] = oh
    ot_ref[...] = ot


def kernel(hypo_scores, joint_logits, beam_width):
    del beam_width
    bm, ba, m, s, xp = pl.pallas_call(
        _pass1_kernel,
        grid=(_NC,),
        in_specs=[pl.BlockSpec((_B, _C), lambda i: (0, i))],
        out_specs=[
            pl.BlockSpec((_NC, _B), lambda i: (0, 0)),
            pl.BlockSpec((_NC, _B), lambda i: (0, 0)),
            pl.BlockSpec((1, _B), lambda i: (0, 0)),
            pl.BlockSpec((1, _B), lambda i: (0, 0)),
            pl.BlockSpec((_B, _C), lambda i: (0, i)),
        ],
        out_shape=[
            jax.ShapeDtypeStruct((_NC, _B), jnp.float32),
            jax.ShapeDtypeStruct((_NC, _B), jnp.int32),
            jax.ShapeDtypeStruct((1, _B), jnp.float32),
            jax.ShapeDtypeStruct((1, _B), jnp.float32),
            jax.ShapeDtypeStruct((_B, _NC * _C), jnp.float32),
        ],
    )(joint_logits)

    os_, oh, ot = pl.pallas_call(
        _topk_kernel,
        in_specs=[
            pl.BlockSpec(memory_space=pltpu.MemorySpace.VMEM),
            pl.BlockSpec(memory_space=pltpu.MemorySpace.VMEM),
            pl.BlockSpec(memory_space=pltpu.MemorySpace.VMEM),
            pl.BlockSpec(memory_space=pltpu.MemorySpace.VMEM),
            pl.BlockSpec(memory_space=pltpu.MemorySpace.VMEM),
            pl.BlockSpec(memory_space=pl.ANY),
        ],
        out_specs=[
            pl.BlockSpec(memory_space=pltpu.MemorySpace.VMEM),
            pl.BlockSpec(memory_space=pltpu.MemorySpace.VMEM),
            pl.BlockSpec(memory_space=pltpu.MemorySpace.VMEM),
        ],
        out_shape=[
            jax.ShapeDtypeStruct((1, _B), jnp.float32),
            jax.ShapeDtypeStruct((1, _B), jnp.int32),
            jax.ShapeDtypeStruct((1, _B), jnp.int32),
        ],
        scratch_shapes=[
            pltpu.VMEM((8, _C), jnp.float32),
            pltpu.SemaphoreType.DMA,
        ],
    )(bm, ba, m, s, hypo_scores.reshape(1, _B), xp)

    return os_.reshape(_B), oh.reshape(_B), ot.reshape(_B)


# speedup vs baseline: 81.4498x; 1.1912x over previous
import jax
import jax.numpy as jnp
from jax.experimental import pallas as pl
from jax.experimental.pallas import tpu as pltpu

_B = 128
_V = 100000
_VN = _V - 1
_C = 1024
_NC = (_V + _C - 1) // _C
_BIG = 2**30


def _pass1_kernel(x_ref, bm_ref, ba_ref, bm2_ref, ba2_ref, m_ref, s_ref,
                  xp_ref):
    i = pl.program_id(0)
    xp_ref[...] = x_ref[...]
    xt = x_ref[...].T
    col = i * _C + jax.lax.broadcasted_iota(jnp.int32, (_C, _B), 0)
    xn = jnp.where(col < _V, xt, -jnp.inf)
    xc = jnp.where(col < _VN, xt, -jnp.inf)
    bm = jnp.max(xc, axis=0, keepdims=True)
    ba = jnp.min(jnp.where(xc == bm, col, _BIG), axis=0, keepdims=True)
    bm_ref[pl.ds(i, 1), :] = bm
    ba_ref[pl.ds(i, 1), :] = ba
    xc2 = jnp.where(col == ba, -jnp.inf, xc)
    bm2 = jnp.max(xc2, axis=0, keepdims=True)
    ba2 = jnp.min(jnp.where(xc2 == bm2, col, _BIG), axis=0, keepdims=True)
    bm2_ref[pl.ds(i, 1), :] = bm2
    ba2_ref[pl.ds(i, 1), :] = ba2

    @pl.when(i == 0)
    def _():
        m_ref[...] = jnp.full((1, _B), -jnp.inf, jnp.float32)
        s_ref[...] = jnp.zeros((1, _B), jnp.float32)

    m_old = m_ref[...]
    s_old = s_ref[...]
    bmax = jnp.max(xn, axis=0, keepdims=True)
    m_new = jnp.maximum(m_old, bmax)
    s_ref[...] = s_old * jnp.exp(m_old - m_new) + jnp.sum(
        jnp.exp(xn - m_new), axis=0, keepdims=True)
    m_ref[...] = m_new


def _topk_kernel(bm_ref, ba_ref, bm2_ref, ba2_ref, m_ref, s_ref, hypo_ref,
                 joint_ref, os_ref, oh_ref, ot_ref, chunk_ref, sem):
    c = hypo_ref[...] - (m_ref[...] + jnp.log(s_ref[...]))
    cand0 = bm_ref[...] + c
    ba0 = ba_ref[...]
    nxt0 = bm2_ref[...] + c
    nxtcol0 = ba2_ref[...]
    rank2d = (jax.lax.broadcasted_iota(jnp.int32, (_NC, _B), 1) * _NC
              + jax.lax.broadcasted_iota(jnp.int32, (_NC, _B), 0))
    lane = jax.lax.broadcasted_iota(jnp.int32, (1, _B), 1)
    ccols = jax.lax.broadcasted_iota(jnp.int32, (1, _C), 1)

    def body(k, carry):
        cand, ba, nxt, nxtcol, have2, os_, oh, ot = carry
        vmax = jnp.max(cand)
        r = jnp.min(jnp.where(cand == vmax, rank2d, _BIG))
        b = r // _NC
        ch = r % _NC
        colsel = jnp.sum(jnp.where(rank2d == r, ba, 0))
        os_ = jnp.where(lane == k, vmax, os_)
        oh = jnp.where(lane == k, b, oh)
        ot = jnp.where(lane == k, colsel, ot)

        need_dma = jnp.max(jnp.where(rank2d == r, have2, 0)) == 0
        b8 = (b // 8) * 8

        @pl.when(need_dma)
        def _():
            copy = pltpu.make_async_copy(
                joint_ref.at[pl.ds(b8, 8), pl.ds(ch * _C, _C)], chunk_ref,
                sem)
            copy.start()
            copy.wait()

        cb = jnp.sum(jnp.where(lane == b, c, 0.0))
        cols = ch * _C + ccols
        row = jax.lax.broadcasted_iota(jnp.int32, (8, _C), 0)
        vals = jnp.sum(jnp.where(row == b - b8, chunk_ref[...], 0.0),
                       axis=0, keepdims=True)
        sc = vals + cb
        valid = cols < jnp.minimum((ch + 1) * _C, _VN)
        remaining = valid & ((sc < vmax) | ((sc == vmax) & (cols > colsel)))
        new_bm = jnp.max(jnp.where(remaining, sc, -jnp.inf))
        new_col = jnp.min(jnp.where(remaining & (sc == new_bm), cols, _BIG))
        nv = jnp.max(jnp.where(rank2d == r, nxt, -jnp.inf))
        ncol = jnp.min(jnp.where(rank2d == r, nxtcol, _BIG))
        newv = jnp.where(need_dma, new_bm, nv)
        newc = jnp.where(need_dma, new_col, ncol)
        cand = jnp.where(rank2d == r, newv, cand)
        ba = jnp.where(rank2d == r, newc, ba)
        have2 = jnp.where(rank2d == r, 0, have2)
        return cand, ba, nxt, nxtcol, have2, os_, oh, ot

    init = (cand0, ba0, nxt0, nxtcol0,
            jnp.ones((_NC, _B), jnp.int32),
            jnp.zeros((1, _B), jnp.float32),
            jnp.zeros((1, _B), jnp.int32),
            jnp.zeros((1, _B), jnp.int32))
    _, _, _, _, _, os_, oh, ot = jax.lax.fori_loop(0, _B, body, init)
    os_ref[...] = os_
    oh_ref[...] = oh
    ot_ref[...] = ot


def kernel(hypo_scores, joint_logits, beam_width):
    del beam_width
    bm, ba, bm2, ba2, m, s, xp = pl.pallas_call(
        _pass1_kernel,
        grid=(_NC,),
        in_specs=[pl.BlockSpec((_B, _C), lambda i: (0, i))],
        out_specs=[
            pl.BlockSpec((_NC, _B), lambda i: (0, 0)),
            pl.BlockSpec((_NC, _B), lambda i: (0, 0)),
            pl.BlockSpec((_NC, _B), lambda i: (0, 0)),
            pl.BlockSpec((_NC, _B), lambda i: (0, 0)),
            pl.BlockSpec((1, _B), lambda i: (0, 0)),
            pl.BlockSpec((1, _B), lambda i: (0, 0)),
            pl.BlockSpec((_B, _C), lambda i: (0, i)),
        ],
        out_shape=[
            jax.ShapeDtypeStruct((_NC, _B), jnp.float32),
            jax.ShapeDtypeStruct((_NC, _B), jnp.int32),
            jax.ShapeDtypeStruct((_NC, _B), jnp.float32),
            jax.ShapeDtypeStruct((_NC, _B), jnp.int32),
            jax.ShapeDtypeStruct((1, _B), jnp.float32),
            jax.ShapeDtypeStruct((1, _B), jnp.float32),
            jax.ShapeDtypeStruct((_B, _NC * _C), jnp.float32),
        ],
    )(joint_logits)

    os_, oh, ot = pl.pallas_call(
        _topk_kernel,
        in_specs=[
            pl.BlockSpec(memory_space=pltpu.MemorySpace.VMEM),
            pl.BlockSpec(memory_space=pltpu.MemorySpace.VMEM),
            pl.BlockSpec(memory_space=pltpu.MemorySpace.VMEM),
            pl.BlockSpec(memory_space=pltpu.MemorySpace.VMEM),
            pl.BlockSpec(memory_space=pltpu.MemorySpace.VMEM),
            pl.BlockSpec(memory_space=pltpu.MemorySpace.VMEM),
            pl.BlockSpec(memory_space=pltpu.MemorySpace.VMEM),
            pl.BlockSpec(memory_space=pl.ANY),
        ],
        out_specs=[
            pl.BlockSpec(memory_space=pltpu.MemorySpace.VMEM),
            pl.BlockSpec(memory_space=pltpu.MemorySpace.VMEM),
            pl.BlockSpec(memory_space=pltpu.MemorySpace.VMEM),
        ],
        out_shape=[
            jax.ShapeDtypeStruct((1, _B), jnp.float32),
            jax.ShapeDtypeStruct((1, _B), jnp.int32),
            jax.ShapeDtypeStruct((1, _B), jnp.int32),
        ],
        scratch_shapes=[
            pltpu.VMEM((8, _C), jnp.float32),
            pltpu.SemaphoreType.DMA,
        ],
    )(bm, ba, bm2, ba2, m, s, hypo_scores.reshape(1, _B), xp)

    return os_.reshape(_B), oh.reshape(_B), ot.reshape(_B)
